# Initial kernel scaffold; baseline (speedup 1.0000x reference)
import functools

import jax
import jax.numpy as jnp
from jax import lax
from jax.experimental import pallas as pl
from jax.experimental.pallas import tpu as pltpu
from jax.experimental.pallas import tpu_sc as plsc

B = 16384
F = 26
V = 1000
D = 64

NC = 2
NS = 16
L = 16
NW = NC * NS
TOT = B * F
PER_W = TOT // NW
C = 1664
NCHUNK = PER_W // C
KS = C // 128
VR = 128 // L


def _make_gather():
    mesh = plsc.VectorSubcoreMesh(core_axis_name="c", subcore_axis_name="s")

    @functools.partial(
        pl.kernel,
        mesh=mesh,
        out_type=jax.ShapeDtypeStruct((TOT, D), jnp.float32),
        scratch_types=[
            pltpu.VMEM((KS, 128), jnp.int32),
            pltpu.VMEM((KS, 128), jnp.int32),
            pltpu.VMEM((C, D), jnp.float32),
            pltpu.SemaphoreType.DMA,
        ],
    )
    def gather_kernel(table_hbm, ids_hbm, out_hbm, raw_v, idx_v, rows_v, sem):
        wid = lax.axis_index("s") * NC + lax.axis_index("c")
        wrow = wid * (PER_W // 128)

        def chunk_body(ci, carry):
            row0 = wrow + ci * KS
            pltpu.sync_copy(ids_hbm.at[pl.ds(row0, KS)], raw_v)

            def off_body(r, c2):
                for q in range(VR):
                    lane = lax.iota(jnp.int32, L)
                    pos = r * 128 + q * L + lane
                    offs = (pos % F) * V
                    sl = pl.ds(q * L, L)
                    idx_v[r, sl] = raw_v[r, sl] + offs
                return c2

            lax.fori_loop(0, KS, off_body, 0)

            copies = []
            for k in range(KS):
                copies.append(
                    pltpu.async_copy(
                        table_hbm.at[idx_v.at[k]],
                        rows_v.at[pl.ds(k * 128, 128)],
                        sem,
                    )
                )
            for cp in copies:
                cp.wait()

            pltpu.sync_copy(rows_v, out_hbm.at[pl.ds(row0 * 128, C)])
            return carry

        lax.fori_loop(0, NCHUNK, chunk_body, 0)

    return gather_kernel


_gather = _make_gather()


@jax.jit
def kernel(inputs, tables):
    ids_flat = inputs.reshape(TOT // 128, 128)
    table_flat = tables.reshape(F * V, D)
    out = _gather(table_flat, ids_flat)
    return out.reshape(B, F, D)


# speedup vs baseline: 4.5822x; 4.5822x over previous
import functools

import jax
import jax.numpy as jnp
from jax import lax
from jax.experimental import pallas as pl
from jax.experimental.pallas import tpu as pltpu
from jax.experimental.pallas import tpu_sc as plsc

B = 16384
F = 26
V = 1000
D = 64

NC = 2
NS = 16
L = 16
NW = NC * NS
TOT = B * F
PER_W = TOT // NW
C = 1664
NCHUNK = PER_W // C
KS = C // 128
VR = 128 // L


def _make_gather():
    mesh = plsc.VectorSubcoreMesh(core_axis_name="c", subcore_axis_name="s")

    @functools.partial(
        pl.kernel,
        mesh=mesh,
        out_type=jax.ShapeDtypeStruct((TOT, D), jnp.float32),
        compiler_params=pltpu.CompilerParams(use_tc_tiling_on_sc=False),
        scratch_types=[
            pltpu.VMEM((C,), jnp.int32),
            pltpu.VMEM((KS, 128), jnp.int32),
            pltpu.VMEM((C, D), jnp.float32),
            pltpu.SemaphoreType.DMA,
        ],
    )
    def gather_kernel(table_hbm, ids_hbm, out_hbm, raw_v, idx_v, rows_v, sem):
        wid = lax.axis_index("s") * NC + lax.axis_index("c")
        wbase = wid * PER_W

        def chunk_body(ci, carry):
            base = wbase + ci * C
            pltpu.sync_copy(ids_hbm.at[pl.ds(base, C)], raw_v)

            def off_body(r, c2):
                for q in range(VR):
                    lane = lax.iota(jnp.int32, L)
                    pos = r * 128 + q * L + lane
                    offs = (pos % F) * V
                    idx_v[r, pl.ds(q * L, L)] = raw_v[pl.ds(r * 128 + q * L, L)] + offs
                return c2

            lax.fori_loop(0, KS, off_body, 0)

            copies = []
            for k in range(KS):
                copies.append(
                    pltpu.async_copy(
                        table_hbm.at[idx_v.at[k]],
                        rows_v.at[pl.ds(k * 128, 128)],
                        sem,
                    )
                )
            for cp in copies:
                cp.wait()

            pltpu.sync_copy(rows_v, out_hbm.at[pl.ds(base, C)])
            return carry

        lax.fori_loop(0, NCHUNK, chunk_body, 0)

    return gather_kernel


_gather = _make_gather()


@jax.jit
def kernel(inputs, tables):
    ids_flat = inputs.reshape(TOT)
    table_flat = tables.reshape(F * V, D)
    out = _gather(table_flat, ids_flat)
    return out.reshape(B, F, D)


# speedup vs baseline: 4.6276x; 1.0099x over previous
import functools

import jax
import jax.numpy as jnp
from jax import lax
from jax.experimental import pallas as pl
from jax.experimental.pallas import tpu as pltpu
from jax.experimental.pallas import tpu_sc as plsc

B = 16384
F = 26
V = 1000
D = 64

NC = 2
NS = 16
L = 16
NW = NC * NS
TOT = B * F
PER_W = TOT // NW
C = 512
NCHUNK = PER_W // C
NPAIR = NCHUNK // 2
KS = C // 128
VR = 128 // L


def _make_gather():
    mesh = plsc.VectorSubcoreMesh(core_axis_name="c", subcore_axis_name="s")

    @functools.partial(
        pl.kernel,
        mesh=mesh,
        out_type=jax.ShapeDtypeStruct((TOT, D), jnp.float32),
        compiler_params=pltpu.CompilerParams(use_tc_tiling_on_sc=False),
        scratch_types=[
            pltpu.VMEM((C,), jnp.int32),
            pltpu.VMEM((KS, 128), jnp.int32),
            pltpu.VMEM((KS, 128), jnp.int32),
            pltpu.VMEM((C, D), jnp.float32),
            pltpu.VMEM((C, D), jnp.float32),
            pltpu.SemaphoreType.DMA,
            pltpu.SemaphoreType.DMA,
            pltpu.SemaphoreType.DMA,
            pltpu.SemaphoreType.DMA,
        ],
    )
    def gather_kernel(table_hbm, ids_hbm, out_hbm,
                      raw_v, idx0_v, idx1_v, rows0_v, rows1_v,
                      sem_g0, sem_g1, sem_o0, sem_o1):
        wid = lax.axis_index("s") * NC + lax.axis_index("c")
        wbase = wid * PER_W

        def load_and_compute(base, idx_v):
            pltpu.sync_copy(ids_hbm.at[pl.ds(base, C)], raw_v)

            def off_body(r, c2):
                for q in range(VR):
                    lane = lax.iota(jnp.int32, L)
                    pos = base + r * 128 + q * L + lane
                    offs = (pos % F) * V
                    idx_v[r, pl.ds(q * L, L)] = (
                        raw_v[pl.ds(r * 128 + q * L, L)] + offs
                    )
                return c2

            lax.fori_loop(0, KS, off_body, 0)

        def fire_gather(idx_v, rows_v, sem):
            for k in range(KS):
                pltpu.async_copy(
                    table_hbm.at[idx_v.at[k]],
                    rows_v.at[pl.ds(k * 128, 128)],
                    sem,
                )

        def wait_gather(idx_v, rows_v, sem):
            for k in range(KS):
                pltpu.make_async_copy(
                    table_hbm.at[idx_v.at[k]],
                    rows_v.at[pl.ds(k * 128, 128)],
                    sem,
                ).wait()

        def fire_wb(base, rows_v, sem):
            pltpu.async_copy(rows_v, out_hbm.at[pl.ds(base, C)], sem)

        def wait_wb(base, rows_v, sem):
            pltpu.make_async_copy(
                rows_v, out_hbm.at[pl.ds(base, C)], sem
            ).wait()

        load_and_compute(wbase, idx0_v)
        fire_gather(idx0_v, rows0_v, sem_g0)
        load_and_compute(wbase + C, idx1_v)
        fire_gather(idx1_v, rows1_v, sem_g1)

        def pair_body(jj, carry):
            base_a = wbase + (2 * jj) * C
            base_b = base_a + C

            wait_gather(idx0_v, rows0_v, sem_g0)
            fire_wb(base_a, rows0_v, sem_o0)

            @pl.when(jj < NPAIR - 1)
            def _():
                load_and_compute(base_a + 2 * C, idx0_v)
                wait_wb(base_a, rows0_v, sem_o0)
                fire_gather(idx0_v, rows0_v, sem_g0)

            wait_gather(idx1_v, rows1_v, sem_g1)
            fire_wb(base_b, rows1_v, sem_o1)

            @pl.when(jj < NPAIR - 1)
            def _():
                load_and_compute(base_b + 2 * C, idx1_v)
                wait_wb(base_b, rows1_v, sem_o1)
                fire_gather(idx1_v, rows1_v, sem_g1)

            return carry

        lax.fori_loop(0, NPAIR, pair_body, 0)

        last_a = wbase + (NCHUNK - 2) * C
        wait_wb(last_a, rows0_v, sem_o0)
        wait_wb(last_a + C, rows1_v, sem_o1)

    return gather_kernel


_gather = _make_gather()


@jax.jit
def kernel(inputs, tables):
    ids_flat = inputs.reshape(TOT)
    table_flat = tables.reshape(F * V, D)
    out = _gather(table_flat, ids_flat)
    return out.reshape(B, F, D)


# speedup vs baseline: 5.9378x; 1.2831x over previous
import functools

import jax
import jax.numpy as jnp
from jax import lax
from jax.experimental import pallas as pl
from jax.experimental.pallas import tpu as pltpu
from jax.experimental.pallas import tpu_sc as plsc

B = 16384
F = 26
V = 1000
D = 64

NC = 2
NS = 16
L = 16
NW = NC * NS
NSPLIT = NW - F
CB = 256
G = CB // L
HALF = B // 2


def _make_lookup():
    mesh = plsc.VectorSubcoreMesh(core_axis_name="c", subcore_axis_name="s")

    @functools.partial(
        pl.kernel,
        mesh=mesh,
        out_type=jax.ShapeDtypeStruct((F, D, B), jnp.float32),
        compiler_params=pltpu.CompilerParams(
            use_tc_tiling_on_sc=True, needs_layout_passes=False),
        scratch_types=[
            pltpu.VMEM((D, V), jnp.float32),
            pltpu.VMEM((B,), jnp.int32),
            pltpu.VMEM((D, CB), jnp.float32),
            pltpu.VMEM((D, CB), jnp.float32),
            pltpu.SemaphoreType.DMA,
            pltpu.SemaphoreType.DMA,
        ],
    )
    def lookup_kernel(table_hbm, ids_hbm, out_hbm,
                      table_v, ids_v, out0_v, out1_v, sem0, sem1):
        wid = lax.axis_index("s") * NC + lax.axis_index("c")

        split = wid < 2 * NSPLIT
        f = jnp.where(split, wid // 2, wid - NSPLIT)
        b0 = jnp.where(split, (wid % 2) * HALF, 0)
        nb = jnp.where(split, HALF, B)
        npair = nb // (2 * CB)

        pltpu.sync_copy(table_hbm.at[f], table_v)
        pltpu.sync_copy(ids_hbm.at[f, pl.ds(b0, HALF)], ids_v.at[pl.ds(0, HALF)])

        @pl.when(jnp.logical_not(split))
        def _():
            pltpu.sync_copy(ids_hbm.at[f, pl.ds(b0 + HALF, HALF)],
                            ids_v.at[pl.ds(HALF, HALF)])

        def compute_chunk(bl, out_v):
            def grp_body(g, carry):
                idvec = ids_v[pl.ds(bl + g * L, L)]
                for d in range(D):
                    dvec = jnp.full((L,), d, jnp.int32)
                    out_v[d, pl.ds(g * L, L)] = plsc.load_gather(
                        table_v, [dvec, idvec])
                return carry

            lax.fori_loop(0, G, grp_body, 0)

        def fire_wb(bl, out_v, sem):
            pltpu.async_copy(
                out_v, out_hbm.at[f, :, pl.ds(b0 + bl, CB)], sem)

        def wait_wb(bl, out_v, sem):
            pltpu.make_async_copy(
                out_v, out_hbm.at[f, :, pl.ds(b0 + bl, CB)], sem).wait()

        compute_chunk(0, out0_v)
        fire_wb(0, out0_v, sem0)
        compute_chunk(CB, out1_v)
        fire_wb(CB, out1_v, sem1)

        def pair_loop(jj, carry):
            bl_a = (2 * jj + 2) * CB
            bl_b = bl_a + CB

            @pl.when(jj < npair - 1)
            def _():
                wait_wb(bl_a - 2 * CB, out0_v, sem0)
                compute_chunk(bl_a, out0_v)
                fire_wb(bl_a, out0_v, sem0)
                wait_wb(bl_b - 2 * CB, out1_v, sem1)
                compute_chunk(bl_b, out1_v)
                fire_wb(bl_b, out1_v, sem1)

            return carry

        lax.fori_loop(0, B // (2 * CB) - 1, pair_loop, 0)

        last_a = nb - 2 * CB
        wait_wb(last_a, out0_v, sem0)
        wait_wb(last_a + CB, out1_v, sem1)

    return lookup_kernel


_lookup = _make_lookup()


@jax.jit
def kernel(inputs, tables):
    ids_t = inputs.T
    table_t = jnp.transpose(tables, (0, 2, 1))
    out_t = _lookup(table_t, ids_t)
    return jnp.transpose(out_t, (2, 0, 1))


# speedup vs baseline: 12.4721x; 2.1005x over previous
import functools

import jax
import jax.numpy as jnp
from jax import lax
from jax.experimental import pallas as pl
from jax.experimental.pallas import tpu as pltpu
from jax.experimental.pallas import tpu_sc as plsc

B = 16384
F = 26
V = 1000
D = 64

NC = 2
NS = 16
L = 16
NW = NC * NS
NSPLIT = NW - F
CB = 256
G = CB // L
KU = 8
HALF = B // 2


def _make_lookup():
    mesh = plsc.VectorSubcoreMesh(core_axis_name="c", subcore_axis_name="s")

    @functools.partial(
        pl.kernel,
        mesh=mesh,
        out_type=jax.ShapeDtypeStruct((F, D, B), jnp.float32),
        compiler_params=pltpu.CompilerParams(
            use_tc_tiling_on_sc=True, needs_layout_passes=False),
        scratch_types=[
            pltpu.VMEM((D, V), jnp.float32),
            pltpu.VMEM((B,), jnp.int32),
            pltpu.VMEM((D, CB), jnp.float32),
            pltpu.VMEM((D, CB), jnp.float32),
            pltpu.SemaphoreType.DMA,
            pltpu.SemaphoreType.DMA,
        ],
    )
    def lookup_kernel(table_hbm, ids_hbm, out_hbm,
                      table_v, ids_v, out0_v, out1_v, sem0, sem1):
        wid = lax.axis_index("s") * NC + lax.axis_index("c")

        split = wid < 2 * NSPLIT
        f = jnp.where(split, wid // 2, wid - NSPLIT)
        b0 = jnp.where(split, (wid % 2) * HALF, 0)
        nb = jnp.where(split, HALF, B)
        npair = nb // (2 * CB)

        pltpu.sync_copy(table_hbm.at[f], table_v)
        pltpu.sync_copy(ids_hbm.at[f, pl.ds(b0, HALF)], ids_v.at[pl.ds(0, HALF)])

        @pl.when(jnp.logical_not(split))
        def _():
            pltpu.sync_copy(ids_hbm.at[f, pl.ds(b0 + HALF, HALF)],
                            ids_v.at[pl.ds(HALF, HALF)])

        def compute_chunk(bl, out_v):
            def grp_body(g, carry):
                idvec = ids_v[pl.ds(bl + g * L, L)]
                for d0 in range(0, D, KU):
                    vals = [
                        plsc.load_gather(
                            table_v, [jnp.full((L,), d, jnp.int32), idvec])
                        for d in range(d0, d0 + KU)
                    ]
                    for j in range(KU):
                        out_v[d0 + j, pl.ds(g * L, L)] = vals[j]
                return carry

            lax.fori_loop(0, G, grp_body, 0)

        def fire_wb(bl, out_v, sem):
            pltpu.async_copy(
                out_v, out_hbm.at[f, :, pl.ds(b0 + bl, CB)], sem)

        def wait_wb(bl, out_v, sem):
            pltpu.make_async_copy(
                out_v, out_hbm.at[f, :, pl.ds(b0 + bl, CB)], sem).wait()

        compute_chunk(0, out0_v)
        fire_wb(0, out0_v, sem0)
        compute_chunk(CB, out1_v)
        fire_wb(CB, out1_v, sem1)

        def pair_loop(jj, carry):
            bl_a = (2 * jj + 2) * CB
            bl_b = bl_a + CB

            @pl.when(jj < npair - 1)
            def _():
                wait_wb(bl_a - 2 * CB, out0_v, sem0)
                compute_chunk(bl_a, out0_v)
                fire_wb(bl_a, out0_v, sem0)
                wait_wb(bl_b - 2 * CB, out1_v, sem1)
                compute_chunk(bl_b, out1_v)
                fire_wb(bl_b, out1_v, sem1)

            return carry

        lax.fori_loop(0, B // (2 * CB) - 1, pair_loop, 0)

        last_a = nb - 2 * CB
        wait_wb(last_a, out0_v, sem0)
        wait_wb(last_a + CB, out1_v, sem1)

    return lookup_kernel


_lookup = _make_lookup()


@jax.jit
def kernel(inputs, tables):
    ids_t = inputs.T
    table_t = jnp.transpose(tables, (0, 2, 1))
    out_t = _lookup(table_t, ids_t)
    return jnp.transpose(out_t, (2, 0, 1))


# speedup vs baseline: 13.1324x; 1.0529x over previous
import functools

import jax
import jax.numpy as jnp
from jax import lax
from jax.experimental import pallas as pl
from jax.experimental.pallas import tpu as pltpu
from jax.experimental.pallas import tpu_sc as plsc

B = 16384
F = 26
V = 1000
D = 64

NC = 2
NS = 16
L = 16
NW = NC * NS
NSPLIT = NW - F
CB = 256
G = CB // L
KU = 8
HALF = B // 2


def _make_lookup():
    mesh = plsc.VectorSubcoreMesh(core_axis_name="c", subcore_axis_name="s")

    @functools.partial(
        pl.kernel,
        mesh=mesh,
        out_type=jax.ShapeDtypeStruct((F, D, B), jnp.float32),
        compiler_params=pltpu.CompilerParams(
            use_tc_tiling_on_sc=True, needs_layout_passes=False),
        scratch_types=[
            pltpu.VMEM((D, V), jnp.float32),
            pltpu.VMEM((B,), jnp.int32),
            pltpu.VMEM((D, CB), jnp.float32),
            pltpu.VMEM((D, CB), jnp.float32),
            pltpu.SemaphoreType.DMA,
            pltpu.SemaphoreType.DMA,
        ],
    )
    def lookup_kernel(table_hbm, ids_hbm, out_hbm,
                      table_v, ids_v, out0_v, out1_v, sem0, sem1):
        wid = lax.axis_index("s") * NC + lax.axis_index("c")

        split = wid < 2 * NSPLIT
        f = jnp.where(split, wid // 2, wid - NSPLIT)
        b0 = jnp.where(split, (wid % 2) * HALF, 0)
        nb = jnp.where(split, HALF, B)
        npair = nb // (2 * CB)

        pltpu.sync_copy(table_hbm.at[f], table_v)
        pltpu.sync_copy(ids_hbm.at[f, pl.ds(b0, HALF)], ids_v.at[pl.ds(0, HALF)])

        @pl.when(jnp.logical_not(split))
        def _():
            pltpu.sync_copy(ids_hbm.at[f, pl.ds(b0 + HALF, HALF)],
                            ids_v.at[pl.ds(HALF, HALF)])

        def compute_chunk(bl, out_v):
            def grp_body(g, carry):
                idvec = ids_v[pl.ds(bl + g * L, L)]

                def burst(d0):
                    return [
                        plsc.load_gather(
                            table_v, [jnp.full((L,), d, jnp.int32), idvec])
                        for d in range(d0, d0 + KU)
                    ]

                vals = burst(0)
                for d0 in range(KU, D, KU):
                    nvals = burst(d0)
                    for j in range(KU):
                        out_v[d0 - KU + j, pl.ds(g * L, L)] = vals[j]
                    vals = nvals
                for j in range(KU):
                    out_v[D - KU + j, pl.ds(g * L, L)] = vals[j]
                return carry

            lax.fori_loop(0, G, grp_body, 0)

        def fire_wb(bl, out_v, sem):
            pltpu.async_copy(
                out_v, out_hbm.at[f, :, pl.ds(b0 + bl, CB)], sem)

        def wait_wb(bl, out_v, sem):
            pltpu.make_async_copy(
                out_v, out_hbm.at[f, :, pl.ds(b0 + bl, CB)], sem).wait()

        compute_chunk(0, out0_v)
        fire_wb(0, out0_v, sem0)
        compute_chunk(CB, out1_v)
        fire_wb(CB, out1_v, sem1)

        def pair_loop(jj, carry):
            bl_a = (2 * jj + 2) * CB
            bl_b = bl_a + CB

            @pl.when(jj < npair - 1)
            def _():
                wait_wb(bl_a - 2 * CB, out0_v, sem0)
                compute_chunk(bl_a, out0_v)
                fire_wb(bl_a, out0_v, sem0)
                wait_wb(bl_b - 2 * CB, out1_v, sem1)
                compute_chunk(bl_b, out1_v)
                fire_wb(bl_b, out1_v, sem1)

            return carry

        lax.fori_loop(0, B // (2 * CB) - 1, pair_loop, 0)

        last_a = nb - 2 * CB
        wait_wb(last_a, out0_v, sem0)
        wait_wb(last_a + CB, out1_v, sem1)

    return lookup_kernel


_lookup = _make_lookup()


@jax.jit
def kernel(inputs, tables):
    ids_t = inputs.T
    table_t = jnp.transpose(tables, (0, 2, 1))
    out_t = _lookup(table_t, ids_t)
    return jnp.transpose(out_t, (2, 0, 1))


# speedup vs baseline: 15.1293x; 1.1521x over previous
import functools

import jax
import jax.numpy as jnp
from jax import lax
from jax.experimental import pallas as pl
from jax.experimental.pallas import tpu as pltpu
from jax.experimental.pallas import tpu_sc as plsc

B = 16384
F = 26
V = 1000
D = 64

NC = 2
NS = 16
L = 16
NW = NC * NS
NSPLIT = NW - F
CB = 256
G = CB // L
KU = 8
HALF = B // 2


def _make_lookup():
    mesh = plsc.VectorSubcoreMesh(core_axis_name="c", subcore_axis_name="s")

    @functools.partial(
        pl.kernel,
        mesh=mesh,
        out_type=jax.ShapeDtypeStruct((F, D, B), jnp.float32),
        compiler_params=pltpu.CompilerParams(
            use_tc_tiling_on_sc=True, needs_layout_passes=False),
        scratch_types=[
            pltpu.VMEM((D, V), jnp.float32),
            pltpu.VMEM((B,), jnp.int32),
            pltpu.VMEM((D, CB), jnp.float32),
            pltpu.VMEM((D, CB), jnp.float32),
            pltpu.SemaphoreType.DMA,
            pltpu.SemaphoreType.DMA,
        ],
    )
    def lookup_kernel(table_hbm, ids_hbm, out_hbm,
                      table_v, ids_v, out0_v, out1_v, sem0, sem1):
        wid = lax.axis_index("s") * NC + lax.axis_index("c")

        split = wid < 2 * NSPLIT
        f = jnp.where(split, wid // 2, wid - NSPLIT)
        b0 = jnp.where(split, (wid % 2) * HALF, 0)
        nb = jnp.where(split, HALF, B)
        npair = nb // (2 * CB)

        pltpu.sync_copy(table_hbm.at[f], table_v)
        pltpu.sync_copy(ids_hbm.at[f, pl.ds(b0, HALF)], ids_v.at[pl.ds(0, HALF)])

        @pl.when(jnp.logical_not(split))
        def _():
            pltpu.sync_copy(ids_hbm.at[f, pl.ds(b0 + HALF, HALF)],
                            ids_v.at[pl.ds(HALF, HALF)])

        def compute_chunk(bl, out_v):
            @plsc.parallel_loop(0, G, 1, unroll=1)
            def grp_body(g):
                idvec = ids_v[pl.ds(bl + g * L, L)]

                def burst(d0):
                    return [
                        plsc.load_gather(
                            table_v, [jnp.full((L,), d, jnp.int32), idvec])
                        for d in range(d0, d0 + KU)
                    ]

                vals = burst(0)
                for d0 in range(KU, D, KU):
                    nvals = burst(d0)
                    for j in range(KU):
                        out_v[d0 - KU + j, pl.ds(g * L, L)] = vals[j]
                    vals = nvals
                for j in range(KU):
                    out_v[D - KU + j, pl.ds(g * L, L)] = vals[j]

        def fire_wb(bl, out_v, sem):
            pltpu.async_copy(
                out_v, out_hbm.at[f, :, pl.ds(b0 + bl, CB)], sem)

        def wait_wb(bl, out_v, sem):
            pltpu.make_async_copy(
                out_v, out_hbm.at[f, :, pl.ds(b0 + bl, CB)], sem).wait()

        compute_chunk(0, out0_v)
        fire_wb(0, out0_v, sem0)
        compute_chunk(CB, out1_v)
        fire_wb(CB, out1_v, sem1)

        def pair_loop(jj, carry):
            bl_a = (2 * jj + 2) * CB
            bl_b = bl_a + CB

            @pl.when(jj < npair - 1)
            def _():
                wait_wb(bl_a - 2 * CB, out0_v, sem0)
                compute_chunk(bl_a, out0_v)
                fire_wb(bl_a, out0_v, sem0)
                wait_wb(bl_b - 2 * CB, out1_v, sem1)
                compute_chunk(bl_b, out1_v)
                fire_wb(bl_b, out1_v, sem1)

            return carry

        lax.fori_loop(0, B // (2 * CB) - 1, pair_loop, 0)

        last_a = nb - 2 * CB
        wait_wb(last_a, out0_v, sem0)
        wait_wb(last_a + CB, out1_v, sem1)

    return lookup_kernel


_lookup = _make_lookup()


@jax.jit
def kernel(inputs, tables):
    ids_t = inputs.T
    table_t = jnp.transpose(tables, (0, 2, 1))
    out_t = _lookup(table_t, ids_t)
    return jnp.transpose(out_t, (2, 0, 1))


# speedup vs baseline: 16.6200x; 1.0985x over previous
import functools

import jax
import jax.numpy as jnp
from jax import lax
from jax.experimental import pallas as pl
from jax.experimental.pallas import tpu as pltpu
from jax.experimental.pallas import tpu_sc as plsc

B = 16384
F = 26
V = 1000
D = 64

NC = 2
NS = 16
L = 16
NW = NC * NS
PER_T = B * F // NW
SEG = 1024
NSEG = PER_T // SEG
CB = 256
G = CB // L
KU = 8
MAXPAIR = PER_T // (2 * CB)


def _make_lookup():
    mesh = plsc.VectorSubcoreMesh(core_axis_name="c", subcore_axis_name="s")

    @functools.partial(
        pl.kernel,
        mesh=mesh,
        out_type=jax.ShapeDtypeStruct((F, D, B), jnp.float32),
        compiler_params=pltpu.CompilerParams(
            use_tc_tiling_on_sc=True, needs_layout_passes=False),
        scratch_types=[
            pltpu.VMEM((D, V), jnp.float32),
            pltpu.VMEM((PER_T,), jnp.int32),
            pltpu.VMEM((D, CB), jnp.float32),
            pltpu.VMEM((D, CB), jnp.float32),
            pltpu.SemaphoreType.DMA,
            pltpu.SemaphoreType.DMA,
            pltpu.SemaphoreType.DMA,
        ],
    )
    def lookup_kernel(table_hbm, ids_hbm, out_hbm,
                      table_v, ids_v, out0_v, out1_v, sem_i, sem0, sem1):
        wid = lax.axis_index("s") * NC + lax.axis_index("c")

        w0 = wid * PER_T
        fa = w0 // B
        ba = w0 % B
        la = jnp.minimum(B - ba, PER_T)
        lb = PER_T - la
        fb = jnp.minimum(fa + 1, F - 1)

        def seg_copy(s):
            return pltpu.async_copy(
                ids_hbm.at[fa, pl.ds(ba + s * SEG, SEG)],
                ids_v.at[pl.ds(s * SEG, SEG)], sem_i)

        def seg_copy_b(s):
            return pltpu.async_copy(
                ids_hbm.at[fb, pl.ds(s * SEG, SEG)],
                ids_v.at[pl.ds(la + s * SEG, SEG)], sem_i)

        for s in range(NSEG):
            @pl.when(s * SEG < la)
            def _():
                seg_copy(s)

            @pl.when(s * SEG < lb)
            def _():
                seg_copy_b(s)

        for s in range(NSEG):
            @pl.when(s * SEG < la)
            def _():
                pltpu.make_async_copy(
                    ids_hbm.at[fa, pl.ds(ba + s * SEG, SEG)],
                    ids_v.at[pl.ds(s * SEG, SEG)], sem_i).wait()

            @pl.when(s * SEG < lb)
            def _():
                pltpu.make_async_copy(
                    ids_hbm.at[fb, pl.ds(s * SEG, SEG)],
                    ids_v.at[pl.ds(la + s * SEG, SEG)], sem_i).wait()

        def compute_chunk(wl, out_v):
            @plsc.parallel_loop(0, G, 1, unroll=1)
            def grp_body(g):
                idvec = ids_v[pl.ds(wl + g * L, L)]

                def burst(d0):
                    return [
                        plsc.load_gather(
                            table_v, [jnp.full((L,), d, jnp.int32), idvec])
                        for d in range(d0, d0 + KU)
                    ]

                vals = burst(0)
                for d0 in range(KU, D, KU):
                    nvals = burst(d0)
                    for j in range(KU):
                        out_v[d0 - KU + j, pl.ds(g * L, L)] = vals[j]
                    vals = nvals
                for j in range(KU):
                    out_v[D - KU + j, pl.ds(g * L, L)] = vals[j]

        def process_part(f, b_hbm0, wl0, n):
            npair = n // (2 * CB)

            def fire_wb(k, out_v, sem):
                pltpu.async_copy(
                    out_v, out_hbm.at[f, :, pl.ds(b_hbm0 + k * CB, CB)], sem)

            def wait_wb(k, out_v, sem):
                pltpu.make_async_copy(
                    out_v, out_hbm.at[f, :, pl.ds(b_hbm0 + k * CB, CB)],
                    sem).wait()

            pltpu.sync_copy(table_hbm.at[f], table_v)

            compute_chunk(wl0, out0_v)
            fire_wb(0, out0_v, sem0)
            compute_chunk(wl0 + CB, out1_v)
            fire_wb(1, out1_v, sem1)

            def pair_loop(jj, carry):
                ka = 2 * jj + 2

                @pl.when(jj < npair - 1)
                def _():
                    wait_wb(ka - 2, out0_v, sem0)
                    compute_chunk(wl0 + ka * CB, out0_v)
                    fire_wb(ka, out0_v, sem0)
                    wait_wb(ka - 1, out1_v, sem1)
                    compute_chunk(wl0 + (ka + 1) * CB, out1_v)
                    fire_wb(ka + 1, out1_v, sem1)

                return carry

            lax.fori_loop(0, MAXPAIR - 1, pair_loop, 0)

            last = n // CB - 2
            wait_wb(last, out0_v, sem0)
            wait_wb(last + 1, out1_v, sem1)

        process_part(fa, ba, 0, la)

        @pl.when(lb > 0)
        def _():
            process_part(fb, 0, la, lb)

    return lookup_kernel


_lookup = _make_lookup()


@jax.jit
def kernel(inputs, tables):
    ids_t = inputs.T
    table_t = jnp.transpose(tables, (0, 2, 1))
    out_t = _lookup(table_t, ids_t)
    return jnp.transpose(out_t, (2, 0, 1))
